# Initial kernel scaffold; baseline (speedup 1.0000x reference)
import jax
import jax.numpy as jnp
from jax.experimental import pallas as pl


def kernel(r_boxes):
    raise NotImplementedError("write your pallas kernel here")


# speedup vs baseline: 71.7789x; 71.7789x over previous
import functools

import jax
import jax.numpy as jnp
from jax.experimental import pallas as pl
from jax.experimental.pallas import tpu as pltpu

_THR = 0.7
_TOPN = 1000
_EPS = 1e-8
_RB = 8
_CB = 128


def _corners(xc, yc, w, h, th):
    c = jnp.cos(th)
    s = jnp.sin(th)
    dx = w * 0.5
    dy = h * 0.5
    xs, ys = [], []
    for lx, ly in ((-1.0, -1.0), (1.0, -1.0), (1.0, 1.0), (-1.0, 1.0)):
        ox = lx * dx
        oy = ly * dy
        xs.append(xc + ox * c - oy * s)
        ys.append(yc + ox * s + oy * c)
    return xs, ys


def _dir_area(px, py, qx, qy):
    ex = [qx[(p + 1) % 4] - qx[p] for p in range(4)]
    ey = [qy[(p + 1) % 4] - qy[p] for p in range(4)]
    d = [[ex[p] * (py[v] - qy[p]) - ey[p] * (px[v] - qx[p]) for p in range(4)]
         for v in range(4)]
    total = None
    for k in range(4):
        k1 = (k + 1) % 4
        t0 = jnp.zeros_like(d[0][0])
        t1 = jnp.ones_like(d[0][0])
        empty = None
        for p in range(4):
            da, db = d[k][p], d[k1][p]
            denom = da - db
            t = da / jnp.where(denom == 0.0, 1.0, denom)
            t0 = jnp.where((da < 0.0) & (db >= 0.0), jnp.maximum(t0, t), t0)
            t1 = jnp.where((da >= 0.0) & (db < 0.0), jnp.minimum(t1, t), t1)
            both_out = (da < 0.0) & (db < 0.0)
            empty = both_out if empty is None else (empty | both_out)
        t1 = jnp.maximum(t1, t0)
        span = jnp.where(empty, 0.0, t1 - t0)
        contrib = (py[k1] - py[k]) * (
            px[k] * span + (px[k1] - px[k]) * 0.5 * span * (t1 + t0))
        total = contrib if total is None else total + contrib
    return total


def _pair_kernel(n, boxes_ref, boxes_t_ref, m_ref):
    ib = pl.program_id(0)
    jb = pl.program_id(1)
    needed = jb * _CB + (_CB - 1) > ib * _RB

    @pl.when(needed)
    def _():
        b = boxes_ref[...]
        bt = boxes_t_ref[...]
        xc_r, yc_r = b[:, 0:1], b[:, 1:2]
        w_r, h_r, th_r = b[:, 2:3], b[:, 3:4], b[:, 4:5]
        xc_c, yc_c = bt[0:1, :], bt[1:2, :]
        w_c, h_c, th_c = bt[2:3, :], bt[3:4, :], bt[4:5, :]
        rx, ry = _corners(xc_r, yc_r, w_r, h_r, th_r)
        cx, cy = _corners(xc_c, yc_c, w_c, h_c, th_c)
        inter = _dir_area(rx, ry, cx, cy) + _dir_area(cx, cy, rx, ry)
        iou = inter / (w_r * h_r + w_c * h_c - inter + _EPS)
        row_id = ib * _RB + jax.lax.broadcasted_iota(jnp.int32, (_RB, _CB), 0)
        col_id = jb * _CB + jax.lax.broadcasted_iota(jnp.int32, (_RB, _CB), 1)
        valid = (col_id > row_id) & (col_id < n) & (row_id < n)
        m_ref[...] = jnp.where(valid & (iou >= _THR), 1.0, 0.0)

    @pl.when(jnp.logical_not(needed))
    def _():
        m_ref[...] = jnp.zeros_like(m_ref)


def _greedy_kernel(m_ref, mdiag_ref, keep_ref, supp_ref):
    b = pl.program_id(0)

    @pl.when(b == 0)
    def _():
        supp_ref[...] = jnp.zeros_like(supp_ref)

    base = pl.multiple_of(b * 128, 128)
    incoming = supp_ref[:, pl.ds(base, 128)]
    pre = jnp.where(incoming < 0.5, 1.0, 0.0)
    sub = mdiag_ref[...]

    def cond(c):
        return jnp.logical_not(c[1])

    def body(c):
        a, _ = c
        s = jnp.dot(a, sub, preferred_element_type=jnp.float32)
        a2 = jnp.where(s > 0.5, 0.0, pre)
        return a2, jnp.all(a2 == a)

    a, _ = jax.lax.while_loop(cond, body, (pre, jnp.zeros((), jnp.bool_)))
    rows = m_ref[...]
    add = jnp.dot(a, rows, preferred_element_type=jnp.float32)
    supp_ref[...] = jnp.maximum(supp_ref[...], jnp.minimum(add, 1.0))
    keep_ref[:, pl.ds(base, 128)] = a


def _run(r_boxes, interpret=False):
    n = r_boxes.shape[0]
    np_ = ((n + 127) // 128) * 128
    boxes_p = jnp.zeros((np_, 5), jnp.float32).at[:n].set(r_boxes)
    boxes_t = boxes_p.T
    m = pl.pallas_call(
        functools.partial(_pair_kernel, n),
        grid=(np_ // _RB, np_ // _CB),
        in_specs=[pl.BlockSpec((_RB, 5), lambda i, j: (i, 0)),
                  pl.BlockSpec((5, _CB), lambda i, j: (0, j))],
        out_specs=pl.BlockSpec((_RB, _CB), lambda i, j: (i, j)),
        out_shape=jax.ShapeDtypeStruct((np_, np_), jnp.float32),
        compiler_params=pltpu.CompilerParams(
            dimension_semantics=("arbitrary", "arbitrary")),
        interpret=interpret,
    )(boxes_p, boxes_t)
    nb = np_ // 128
    keep = pl.pallas_call(
        _greedy_kernel,
        grid=(nb,),
        in_specs=[pl.BlockSpec((128, np_), lambda b: (b, 0)),
                  pl.BlockSpec((128, 128), lambda b: (b, b))],
        out_specs=pl.BlockSpec((1, np_), lambda b: (0, 0)),
        out_shape=jax.ShapeDtypeStruct((1, np_), jnp.float32),
        scratch_shapes=[pltpu.VMEM((1, np_), jnp.float32)],
        compiler_params=pltpu.CompilerParams(
            dimension_semantics=("arbitrary",)),
        interpret=interpret,
    )(m, m)
    keepb = keep[0, :n] > 0.5
    cum = jnp.cumsum(keepb.astype(jnp.int32))
    sel = keepb & (cum <= _TOPN)
    return jnp.nonzero(sel, size=_TOPN, fill_value=-1)[0].astype(jnp.int64)


def kernel(r_boxes):
    return _run(r_boxes)


# speedup vs baseline: 203.4189x; 2.8340x over previous
import functools

import jax
import jax.numpy as jnp
import numpy as np
from jax.experimental import pallas as pl
from jax.experimental.pallas import tpu as pltpu

_THR = 0.7
_TOPN = 1000
_EPS = 1e-8
_RB = 8
_CB = 256


def _corners(xc, yc, w, h, th):
    c = jnp.cos(th)
    s = jnp.sin(th)
    dx = w * 0.5
    dy = h * 0.5
    xs, ys = [], []
    for lx, ly in ((-1.0, -1.0), (1.0, -1.0), (1.0, 1.0), (-1.0, 1.0)):
        ox = lx * dx
        oy = ly * dy
        xs.append(xc + ox * c - oy * s)
        ys.append(yc + ox * s + oy * c)
    return xs, ys


def _dir_area(px, py, qx, qy):
    ex = [qx[(p + 1) % 4] - qx[p] for p in range(4)]
    ey = [qy[(p + 1) % 4] - qy[p] for p in range(4)]
    cst = [ex[p] * qy[p] - ey[p] * qx[p] for p in range(4)]
    d = [[ex[p] * py[v] - ey[p] * px[v] - cst[p] for p in range(4)]
         for v in range(4)]
    total = None
    for k in range(4):
        k1 = (k + 1) % 4
        t0 = jnp.zeros_like(d[0][0])
        t1 = jnp.ones_like(d[0][0])
        empty = None
        for p in range(4):
            da, db = d[k][p], d[k1][p]
            denom = da - db
            t = da / jnp.where(denom == 0.0, 1.0, denom)
            t0 = jnp.where((da < 0.0) & (db >= 0.0), jnp.maximum(t0, t), t0)
            t1 = jnp.where((da >= 0.0) & (db < 0.0), jnp.minimum(t1, t), t1)
            both_out = (da < 0.0) & (db < 0.0)
            empty = both_out if empty is None else (empty | both_out)
        t1 = jnp.maximum(t1, t0)
        span = jnp.where(empty, 0.0, t1 - t0)
        contrib = (py[k1] - py[k]) * (
            px[k] * span + (px[k1] - px[k]) * 0.5 * span * (t1 + t0))
        total = contrib if total is None else total + contrib
    return total


def _pair_kernel(n, ibm_ref, jbm_ref, boxes_ref, boxes_t_ref, m_ref):
    t = pl.program_id(0)
    ib = ibm_ref[t]
    jb = jbm_ref[t]
    b = boxes_ref[...]
    bt = boxes_t_ref[...]
    xc_r, yc_r = b[:, 0:1], b[:, 1:2]
    w_r, h_r, th_r = b[:, 2:3], b[:, 3:4], b[:, 4:5]
    xc_c, yc_c = bt[0:1, :], bt[1:2, :]
    w_c, h_c, th_c = bt[2:3, :], bt[3:4, :], bt[4:5, :]
    rx, ry = _corners(xc_r, yc_r, w_r, h_r, th_r)
    cx, cy = _corners(xc_c, yc_c, w_c, h_c, th_c)
    inter = _dir_area(rx, ry, cx, cy) + _dir_area(cx, cy, rx, ry)
    iou = inter / (w_r * h_r + w_c * h_c - inter + _EPS)
    row_id = ib * _RB + jax.lax.broadcasted_iota(jnp.int32, (_RB, _CB), 0)
    col_id = jb * _CB + jax.lax.broadcasted_iota(jnp.int32, (_RB, _CB), 1)
    valid = (col_id > row_id) & (col_id < n) & (row_id < n)
    m_ref[...] = jnp.where(valid & (iou >= _THR), 1.0, 0.0)


def _greedy_kernel(m_ref, mdiag_ref, keep_ref, supp_ref):
    b = pl.program_id(0)

    @pl.when(b == 0)
    def _():
        supp_ref[...] = jnp.zeros_like(supp_ref)

    base = pl.multiple_of(b * 128, 128)
    incoming = supp_ref[:, pl.ds(base, 128)]
    pre = jnp.where(incoming < 0.5, 1.0, 0.0)
    sub = mdiag_ref[...]

    def cond(c):
        return jnp.logical_not(c[1])

    def body(c):
        a, _ = c
        s = jnp.dot(a, sub, preferred_element_type=jnp.float32)
        a2 = jnp.where(s > 0.5, 0.0, pre)
        return a2, jnp.all(a2 == a)

    a, _ = jax.lax.while_loop(cond, body, (pre, jnp.zeros((), jnp.bool_)))
    rows = m_ref[...]
    add = jnp.dot(a, rows, preferred_element_type=jnp.float32)
    supp_ref[...] = jnp.maximum(supp_ref[...], jnp.minimum(add, 1.0))
    keep_ref[:, pl.ds(base, 128)] = a


def _run(r_boxes, interpret=False):
    n = r_boxes.shape[0]
    np_ = ((n + 127) // 128) * 128
    boxes_p = jnp.zeros((np_, 5), jnp.float32).at[:n].set(r_boxes)
    boxes_t = boxes_p.T
    nrb, ncb = np_ // _RB, np_ // _CB
    ibs, jbs = np.meshgrid(np.arange(nrb), np.arange(ncb), indexing="ij")
    need = jbs * _CB + (_CB - 1) > ibs * _RB
    ib_map = jnp.asarray(ibs[need].astype(np.int32))
    jb_map = jnp.asarray(jbs[need].astype(np.int32))
    ntiles = int(ib_map.shape[0])
    m = pl.pallas_call(
        functools.partial(_pair_kernel, n),
        grid_spec=pltpu.PrefetchScalarGridSpec(
            num_scalar_prefetch=2,
            grid=(ntiles,),
            in_specs=[
                pl.BlockSpec((_RB, 5), lambda t, ibm, jbm: (ibm[t], 0)),
                pl.BlockSpec((5, _CB), lambda t, ibm, jbm: (0, jbm[t])),
            ],
            out_specs=pl.BlockSpec((_RB, _CB),
                                   lambda t, ibm, jbm: (ibm[t], jbm[t])),
        ),
        out_shape=jax.ShapeDtypeStruct((np_, np_), jnp.float32),
        compiler_params=pltpu.CompilerParams(
            dimension_semantics=("arbitrary",)),
        interpret=interpret,
    )(ib_map, jb_map, boxes_p, boxes_t)
    nb = np_ // 128
    keep = pl.pallas_call(
        _greedy_kernel,
        grid=(nb,),
        in_specs=[pl.BlockSpec((128, np_), lambda b: (b, 0)),
                  pl.BlockSpec((128, 128), lambda b: (b, b))],
        out_specs=pl.BlockSpec((1, np_), lambda b: (0, 0)),
        out_shape=jax.ShapeDtypeStruct((1, np_), jnp.float32),
        scratch_shapes=[pltpu.VMEM((1, np_), jnp.float32)],
        compiler_params=pltpu.CompilerParams(
            dimension_semantics=("arbitrary",)),
        interpret=interpret,
    )(m, m)
    keepb = keep[0, :n] > 0.5
    cum = jnp.cumsum(keepb.astype(jnp.int32))
    sel = keepb & (cum <= _TOPN)
    return jnp.nonzero(sel, size=_TOPN, fill_value=-1)[0].astype(jnp.int64)


def kernel(r_boxes):
    return _run(r_boxes)


# speedup vs baseline: 840.2086x; 4.1304x over previous
import functools

import jax
import jax.numpy as jnp
import numpy as np
from jax import lax
from jax.experimental import pallas as pl
from jax.experimental.pallas import tpu as pltpu
from jax.experimental.pallas import tpu_sc as plsc

_THR = 0.7
_TOPN = 1000
_EPS = 1e-8
_C = 128
_P1R = 64
_P1C = 512
_NW = 32
_SUB = 32
_BAT = 256
_CLR = 16


def _corners(xc, yc, w, h, th):
    c = jnp.cos(th)
    s = jnp.sin(th)
    dx = w * 0.5
    dy = h * 0.5
    xs, ys = [], []
    for lx, ly in ((-1.0, -1.0), (1.0, -1.0), (1.0, 1.0), (-1.0, 1.0)):
        ox = lx * dx
        oy = ly * dy
        xs.append(xc + ox * c - oy * s)
        ys.append(yc + ox * s + oy * c)
    return xs, ys


def _dir_area(px, py, qx, qy):
    ex = [qx[(p + 1) % 4] - qx[p] for p in range(4)]
    ey = [qy[(p + 1) % 4] - qy[p] for p in range(4)]
    cst = [ex[p] * qy[p] - ey[p] * qx[p] for p in range(4)]
    d = [[ex[p] * py[v] - ey[p] * px[v] - cst[p] for p in range(4)]
         for v in range(4)]
    total = None
    for k in range(4):
        k1 = (k + 1) % 4
        t0 = jnp.zeros_like(d[0][0])
        t1 = jnp.ones_like(d[0][0])
        empty = None
        for p in range(4):
            da, db = d[k][p], d[k1][p]
            denom = da - db
            t = da / jnp.where(denom == 0.0, 1.0, denom)
            t0 = jnp.where((da < 0.0) & (db >= 0.0), jnp.maximum(t0, t), t0)
            t1 = jnp.where((da >= 0.0) & (db < 0.0), jnp.minimum(t1, t), t1)
            both_out = (da < 0.0) & (db < 0.0)
            empty = both_out if empty is None else (empty | both_out)
        t1 = jnp.maximum(t1, t0)
        span = jnp.where(empty, 0.0, t1 - t0)
        contrib = (py[k1] - py[k]) * (
            px[k] * span + (px[k1] - px[k]) * 0.5 * span * (t1 + t0))
        total = contrib if total is None else total + contrib
    return total


def _prep_kernel(bt_ref, tbl_ref):
    bt = bt_ref[...]
    xc, yc = bt[0:1], bt[1:2]
    w, h, th = bt[2:3], bt[3:4], bt[4:5]
    c, s = jnp.cos(th), jnp.sin(th)
    extx = jnp.abs(w * 0.5 * c) + jnp.abs(h * 0.5 * s)
    exty = jnp.abs(w * 0.5 * s) + jnp.abs(h * 0.5 * c)
    tbl_ref[...] = jnp.concatenate([xc, yc, w, h, th, extx, exty, w * h],
                                   axis=0)


def _p1_kernel(n, npa, tblt_ref, tbl_ref, wmat_ref, out_ref):
    ib = pl.program_id(0)
    wmat = wmat_ref[...]
    tr = tblt_ref[...]
    xr, yr = tr[:, 0:1], tr[:, 1:2]
    exr, eyr, ar = tr[:, 5:6], tr[:, 6:7], tr[:, 7:8]
    pieces = []
    for jt in range(npa // _P1C):
        tc = tbl_ref[:, pl.ds(jt * _P1C, _P1C)]
        xc_, yc_ = tc[0:1, :], tc[1:2, :]
        exc, eyc, ac = tc[5:6, :], tc[6:7, :], tc[7:8, :]
        ox = (jnp.minimum(xr + exr, xc_ + exc)
              - jnp.maximum(xr - exr, xc_ - exc))
        oy = (jnp.minimum(yr + eyr, yc_ + eyc)
              - jnp.maximum(yr - eyr, yc_ - eyc))
        oxp = jnp.maximum(ox, 0.0)
        oyp = jnp.maximum(oy, 0.0)
        amin = jnp.minimum(ar, ac)
        amax = jnp.maximum(ar, ac)
        good = (oxp * oyp >= 0.41 * (ar + ac)) & (amin >= 0.699 * amax)
        row_id = ib * _P1R + lax.broadcasted_iota(jnp.int32, (_P1R, _P1C), 0)
        col_id = jt * _P1C + lax.broadcasted_iota(jnp.int32, (_P1R, _P1C), 1)
        good = good & (col_id > row_id) & (col_id < n)
        pieces.append(jnp.dot(jnp.where(good, 1.0, 0.0), wmat,
                              preferred_element_type=jnp.float32))
    out_ref[...] = jnp.concatenate(pieces, axis=1).astype(jnp.int32)


def _p2_kernel(npa, n, tbl_hbm, m0p_hbm, cand_hbm, cbox_hbm,
               tbl_v, w_v, cand_v, cbox_v):
    wid = lax.axis_index("s") * 2 + lax.axis_index("c")
    rows_w = npa // _NW
    nsb = rows_w // _SUB
    wpr = npa // 16
    pltpu.sync_copy(tbl_hbm, tbl_v)
    lanes = lax.iota(jnp.int32, 16)
    neg1 = jnp.full((16,), -1, jnp.int32)

    def sub_batch(sb, _):
        row0 = wid * rows_w + sb * _SUB
        pltpu.sync_copy(m0p_hbm.at[pl.ds(row0, _SUB)], w_v)

        def fill(r, _):
            for ch in range(_C // 16):
                cand_v[r, pl.ds(ch * 16, 16)] = neg1
            return 0

        lax.fori_loop(0, _SUB, fill, 0)

        def row_body(r, _):
            rspl = jnp.full((16,), r, jnp.int32)
            base0 = jnp.zeros((16,), jnp.int32)

            def chunk_body(wc, base):
                words = w_v[r, pl.ds(wc * 16, 16)]
                any_w = jnp.max(words)

                def nonempty(base):
                    for l in range(16):
                        wscal = words[l]

                        def have(b, wscal=wscal, l=l):
                            wspl = jnp.full((16,), wscal, jnp.int32)
                            mask = ((wspl >> lanes) & 1) == 1
                            j_ids = (wc * 16 + l) * 16 + lanes
                            cum = plsc.cumsum(mask.astype(jnp.int32))
                            idx = b + cum - 1
                            mask2 = mask & (idx < _C)
                            plsc.store_scatter(cand_v, [rspl, idx], j_ids,
                                               mask=mask2)
                            for p in range(5):
                                pspl = jnp.full((16,), p, jnp.int32)
                                vals = plsc.load_gather(tbl_v, [pspl, j_ids],
                                                        mask=mask2)
                                plsc.store_scatter(cbox_v,
                                                   [rspl, idx + (p * _C)],
                                                   vals, mask=mask2)
                            return b + plsc.all_reduce_population_count(mask)

                        base = lax.cond(wscal != 0, have, lambda b: b, base)
                    return base

                return lax.cond(any_w > 0, nonempty, lambda b: b, base)

            lax.fori_loop(0, wpr // 16, chunk_body, base0)
            return 0

        lax.fori_loop(0, _SUB, row_body, 0)
        pltpu.sync_copy(cand_v, cand_hbm.at[pl.ds(row0, _SUB)])
        pltpu.sync_copy(cbox_v, cbox_hbm.at[pl.ds(row0, _SUB)])
        return 0

    lax.fori_loop(0, nsb, sub_batch, 0)


def _clip_kernel(boxes_ref, cbox_ref, cand_ref, bits_ref):
    b = boxes_ref[...]
    xc_r, yc_r = b[:, 0:1], b[:, 1:2]
    w_r, h_r, th_r = b[:, 2:3], b[:, 3:4], b[:, 4:5]
    cb = cbox_ref[...]
    xc_c, yc_c = cb[:, 0:_C], cb[:, _C:2 * _C]
    w_c, h_c = cb[:, 2 * _C:3 * _C], cb[:, 3 * _C:4 * _C]
    th_c = cb[:, 4 * _C:5 * _C]
    rx, ry = _corners(xc_r, yc_r, w_r, h_r, th_r)
    cx, cy = _corners(xc_c, yc_c, w_c, h_c, th_c)
    inter = _dir_area(rx, ry, cx, cy) + _dir_area(cx, cy, rx, ry)
    iou = inter / (w_r * h_r + w_c * h_c - inter + _EPS)
    cand = cand_ref[...]
    bits_ref[...] = jnp.where((cand >= 0) & (iou >= _THR), 1.0, 0.0)


def _p4_kernel(npa, n, cand_hbm, bits_hbm, out_hbm,
               supp_v, cand_v, bits_v, keep_v):
    wid = lax.axis_index("s") * 2 + lax.axis_index("c")

    @pl.when(wid == 0)
    def _():
        zeros16 = jnp.zeros((16,), jnp.int32)
        neg1 = jnp.full((16,), -1, jnp.int32)
        ones16 = jnp.full((16,), 1, jnp.int32)

        def z(k, _):
            supp_v[pl.ds(k * 16, 16)] = zeros16
            return 0

        lax.fori_loop(0, npa // 16, z, 0)

        def f(k, _):
            keep_v[pl.ds(k * 16, 16)] = neg1
            return 0

        lax.fori_loop(0, 1024 // 16, f, 0)

        lanes = lax.iota(jnp.int32, 16)
        lane0 = lanes == 0

        def batch(bi, cnt):
            row0 = bi * _BAT
            pltpu.sync_copy(cand_hbm.at[pl.ds(row0, _BAT)], cand_v)
            pltpu.sync_copy(bits_hbm.at[pl.ds(row0, _BAT)], bits_v)

            def group(g, cnt):
                for l in range(16):
                    r = g * 16 + l
                    i = row0 + r
                    chunk = supp_v[pl.ds(row0 + g * 16, 16)]
                    live = ((chunk[l] == 0) & (cnt < _TOPN) & (i < n))

                    def do(c, r=r, i=i):
                        plsc.store_scatter(keep_v, [jnp.full((16,), c,
                                                            jnp.int32)],
                                           jnp.full((16,), i, jnp.int32),
                                           mask=lane0)
                        for ch in range(_C // 16):
                            cm = cand_v[r, pl.ds(ch * 16, 16)]
                            bm = bits_v[r, pl.ds(ch * 16, 16)]
                            msk = (bm > 0.5) & (cm >= 0)
                            plsc.store_scatter(supp_v, [cm], ones16, mask=msk)
                        return c + 1

                    cnt = lax.cond(live, do, lambda c: c, cnt)
                return cnt

            return lax.fori_loop(0, _BAT // 16, group, cnt)

        lax.fori_loop(0, npa // _BAT, batch, 0)
        pltpu.sync_copy(keep_v.at[pl.ds(0, _TOPN)], out_hbm)


def _run(r_boxes, interpret=False):
    n = r_boxes.shape[0]
    npa = ((n + 511) // 512) * 512
    boxes_p = jnp.zeros((npa, 5), jnp.float32).at[:n].set(r_boxes)
    boxes_t = boxes_p.T

    tbl = pl.pallas_call(
        _prep_kernel,
        grid=(1,),
        in_specs=[pl.BlockSpec((5, npa), lambda i: (0, 0))],
        out_specs=pl.BlockSpec((8, npa), lambda i: (0, 0)),
        out_shape=jax.ShapeDtypeStruct((8, npa), jnp.float32),
        interpret=interpret,
    )(boxes_t)
    tblt = tbl.T

    wpr = npa // 16
    wnp = np.zeros((_P1C, _P1C // 16), np.float32)
    for cc in range(_P1C):
        wnp[cc, cc // 16] = float(1 << (cc % 16))
    wmat = jnp.asarray(wnp)
    m0p = pl.pallas_call(
        functools.partial(_p1_kernel, n, npa),
        grid=(npa // _P1R,),
        in_specs=[pl.BlockSpec((_P1R, 8), lambda i: (i, 0)),
                  pl.BlockSpec((8, npa), lambda i: (0, 0)),
                  pl.BlockSpec((_P1C, _P1C // 16), lambda i: (0, 0))],
        out_specs=pl.BlockSpec((_P1R, wpr), lambda i: (i, 0)),
        out_shape=jax.ShapeDtypeStruct((npa, wpr), jnp.int32),
        compiler_params=pltpu.CompilerParams(
            dimension_semantics=("arbitrary",)),
        interpret=interpret,
    )(tblt, tbl, wmat)

    mesh = plsc.VectorSubcoreMesh(core_axis_name="c", subcore_axis_name="s",
                                  num_cores=2, num_subcores=16)
    cand, cbox = pl.kernel(
        functools.partial(_p2_kernel, npa, n),
        out_type=(jax.ShapeDtypeStruct((npa, _C), jnp.int32),
                  jax.ShapeDtypeStruct((npa, 5 * _C), jnp.float32)),
        mesh=mesh,
        compiler_params=pltpu.CompilerParams(needs_layout_passes=False),
        scratch_types=[pltpu.VMEM((8, npa), jnp.float32),
                       pltpu.VMEM((_SUB, wpr), jnp.int32),
                       pltpu.VMEM((_SUB, _C), jnp.int32),
                       pltpu.VMEM((_SUB, 5 * _C), jnp.float32)],
        interpret=interpret,
    )(tbl, m0p)

    bits = pl.pallas_call(
        _clip_kernel,
        grid=(npa // _CLR,),
        in_specs=[pl.BlockSpec((_CLR, 5), lambda t: (t, 0)),
                  pl.BlockSpec((_CLR, 5 * _C), lambda t: (t, 0)),
                  pl.BlockSpec((_CLR, _C), lambda t: (t, 0))],
        out_specs=pl.BlockSpec((_CLR, _C), lambda t: (t, 0)),
        out_shape=jax.ShapeDtypeStruct((npa, _C), jnp.float32),
        compiler_params=pltpu.CompilerParams(
            dimension_semantics=("arbitrary",)),
        interpret=interpret,
    )(boxes_p, cbox, cand)

    keep_idx = pl.kernel(
        functools.partial(_p4_kernel, npa, n),
        out_type=jax.ShapeDtypeStruct((_TOPN,), jnp.int32),
        mesh=mesh,
        compiler_params=pltpu.CompilerParams(needs_layout_passes=False),
        scratch_types=[pltpu.VMEM((npa,), jnp.int32),
                       pltpu.VMEM((_BAT, _C), jnp.int32),
                       pltpu.VMEM((_BAT, _C), jnp.float32),
                       pltpu.VMEM((1024,), jnp.int32)],
        interpret=interpret,
    )(cand, bits)

    return keep_idx.astype(jnp.int64)


def kernel(r_boxes):
    return _run(r_boxes)


# speedup vs baseline: 1280.1165x; 1.5236x over previous
import functools

import jax
import jax.numpy as jnp
import numpy as np
from jax import lax
from jax.experimental import pallas as pl
from jax.experimental.pallas import tpu as pltpu
from jax.experimental.pallas import tpu_sc as plsc

_THR = 0.7
_TOPN = 1000
_EPS = 1e-8
_C = 128
_P1R = 64
_P1C = 512
_NW = 32
_SUB = 32
_BAT = 256
_CLR = 16


def _corners(xc, yc, w, h, th):
    c = jnp.cos(th)
    s = jnp.sin(th)
    dx = w * 0.5
    dy = h * 0.5
    xs, ys = [], []
    for lx, ly in ((-1.0, -1.0), (1.0, -1.0), (1.0, 1.0), (-1.0, 1.0)):
        ox = lx * dx
        oy = ly * dy
        xs.append(xc + ox * c - oy * s)
        ys.append(yc + ox * s + oy * c)
    return xs, ys


def _dir_area(px, py, qx, qy):
    ex = [qx[(p + 1) % 4] - qx[p] for p in range(4)]
    ey = [qy[(p + 1) % 4] - qy[p] for p in range(4)]
    cst = [ex[p] * qy[p] - ey[p] * qx[p] for p in range(4)]
    d = [[ex[p] * py[v] - ey[p] * px[v] - cst[p] for p in range(4)]
         for v in range(4)]
    total = None
    for k in range(4):
        k1 = (k + 1) % 4
        t0 = jnp.zeros_like(d[0][0])
        t1 = jnp.ones_like(d[0][0])
        empty = None
        for p in range(4):
            da, db = d[k][p], d[k1][p]
            denom = da - db
            t = da / jnp.where(denom == 0.0, 1.0, denom)
            t0 = jnp.where((da < 0.0) & (db >= 0.0), jnp.maximum(t0, t), t0)
            t1 = jnp.where((da >= 0.0) & (db < 0.0), jnp.minimum(t1, t), t1)
            both_out = (da < 0.0) & (db < 0.0)
            empty = both_out if empty is None else (empty | both_out)
        t1 = jnp.maximum(t1, t0)
        span = jnp.where(empty, 0.0, t1 - t0)
        contrib = (py[k1] - py[k]) * (
            px[k] * span + (px[k1] - px[k]) * 0.5 * span * (t1 + t0))
        total = contrib if total is None else total + contrib
    return total


def _prep_kernel(bt_ref, tbl_ref):
    bt = bt_ref[...]
    xc, yc = bt[0:1], bt[1:2]
    w, h, th = bt[2:3], bt[3:4], bt[4:5]
    c, s = jnp.cos(th), jnp.sin(th)
    extx = jnp.abs(w * 0.5 * c) + jnp.abs(h * 0.5 * s)
    exty = jnp.abs(w * 0.5 * s) + jnp.abs(h * 0.5 * c)
    tbl_ref[...] = jnp.concatenate([xc, yc, w, h, th, extx, exty, w * h],
                                   axis=0)


def _p1_kernel(n, npa, tblt_ref, tbl_ref, wmat_ref, out_ref):
    ib = pl.program_id(0)
    wmat = wmat_ref[...]
    tr = tblt_ref[...]
    xr, yr = tr[:, 0:1], tr[:, 1:2]
    exr, eyr, ar = tr[:, 5:6], tr[:, 6:7], tr[:, 7:8]
    pieces = []
    for jt in range(npa // _P1C):
        tc = tbl_ref[:, pl.ds(jt * _P1C, _P1C)]
        xc_, yc_ = tc[0:1, :], tc[1:2, :]
        exc, eyc, ac = tc[5:6, :], tc[6:7, :], tc[7:8, :]
        ox = (jnp.minimum(xr + exr, xc_ + exc)
              - jnp.maximum(xr - exr, xc_ - exc))
        oy = (jnp.minimum(yr + eyr, yc_ + eyc)
              - jnp.maximum(yr - eyr, yc_ - eyc))
        oxp = jnp.maximum(ox, 0.0)
        oyp = jnp.maximum(oy, 0.0)
        amin = jnp.minimum(ar, ac)
        amax = jnp.maximum(ar, ac)
        good = (oxp * oyp >= 0.41 * (ar + ac)) & (amin >= 0.699 * amax)
        row_id = ib * _P1R + lax.broadcasted_iota(jnp.int32, (_P1R, _P1C), 0)
        col_id = jt * _P1C + lax.broadcasted_iota(jnp.int32, (_P1R, _P1C), 1)
        good = good & (col_id > row_id) & (col_id < n)
        pieces.append(jnp.dot(jnp.where(good, 1.0, 0.0), wmat,
                              preferred_element_type=jnp.float32))
    out_ref[...] = jnp.concatenate(pieces, axis=1).astype(jnp.int32)


def _p2_kernel(npa, n, tbl_hbm, m0p_hbm, cand_hbm, cbox_hbm,
               tbl_v, w_v, cand_v, cbox_v):
    wid = lax.axis_index("s") * 2 + lax.axis_index("c")
    rows_w = npa // _NW
    nsb = rows_w // _SUB
    wpr = npa // 16
    pltpu.sync_copy(tbl_hbm, tbl_v)
    lanes = lax.iota(jnp.int32, 16)
    neg1 = jnp.full((16,), -1, jnp.int32)

    def sub_batch(sb, _):
        row0 = wid * rows_w + sb * _SUB
        pltpu.sync_copy(m0p_hbm.at[pl.ds(row0, _SUB)], w_v)

        def fill(r, _):
            for ch in range(_C // 16):
                cand_v[r, pl.ds(ch * 16, 16)] = neg1
            return 0

        lax.fori_loop(0, _SUB, fill, 0)

        def row_body(r, _):
            i_row = row0 + r
            rspl = jnp.full((16,), r, jnp.int32)
            base0 = jnp.zeros((16,), jnp.int32)

            def chunk_body(wc, base):
                words = w_v[r, pl.ds(wc * 16, 16)]
                any_w = jnp.max(words)

                def nonempty(base):
                    for l in range(16):
                        wscal = words[l]

                        def have(b, wscal=wscal, l=l):
                            wspl = jnp.full((16,), wscal, jnp.int32)
                            mask = ((wspl >> lanes) & 1) == 1
                            j_ids = (wc * 16 + l) * 16 + lanes
                            cum = plsc.cumsum(mask.astype(jnp.int32))
                            idx = b + cum - 1
                            mask2 = mask & (idx < _C)
                            plsc.store_scatter(cand_v, [rspl, idx], j_ids,
                                               mask=mask2)
                            for p in range(5):
                                pspl = jnp.full((16,), p, jnp.int32)
                                vals = plsc.load_gather(tbl_v, [pspl, j_ids],
                                                        mask=mask2)
                                plsc.store_scatter(cbox_v,
                                                   [rspl, idx + (p * _C)],
                                                   vals, mask=mask2)
                            return b + plsc.all_reduce_population_count(mask)

                        base = lax.cond(wscal != 0, have, lambda b: b, base)
                    return base

                return lax.cond(any_w > 0, nonempty, lambda b: b, base)

            lax.fori_loop(i_row >> 8, wpr // 16, chunk_body, base0)
            return 0

        lax.fori_loop(0, _SUB, row_body, 0)
        pltpu.sync_copy(cand_v, cand_hbm.at[pl.ds(row0, _SUB)])
        pltpu.sync_copy(cbox_v, cbox_hbm.at[pl.ds(row0, _SUB)])
        return 0

    lax.fori_loop(0, nsb, sub_batch, 0)


def _clip_kernel(boxes_ref, cbox_ref, cand_ref, bits_ref):
    b = boxes_ref[...]
    xc_r, yc_r = b[:, 0:1], b[:, 1:2]
    w_r, h_r, th_r = b[:, 2:3], b[:, 3:4], b[:, 4:5]
    cb = cbox_ref[...]
    xc_c, yc_c = cb[:, 0:_C], cb[:, _C:2 * _C]
    w_c, h_c = cb[:, 2 * _C:3 * _C], cb[:, 3 * _C:4 * _C]
    th_c = cb[:, 4 * _C:5 * _C]
    rx, ry = _corners(xc_r, yc_r, w_r, h_r, th_r)
    cx, cy = _corners(xc_c, yc_c, w_c, h_c, th_c)
    inter = _dir_area(rx, ry, cx, cy) + _dir_area(cx, cy, rx, ry)
    iou = inter / (w_r * h_r + w_c * h_c - inter + _EPS)
    cand = cand_ref[...]
    bits_ref[...] = jnp.where((cand >= 0) & (iou >= _THR), 1.0, 0.0)


def _p4_kernel(npa, n, cand_hbm, bits_hbm, out_hbm,
               supp_v, cand_v, bits_v, keep_v):
    wid = lax.axis_index("s") * 2 + lax.axis_index("c")

    @pl.when(wid == 0)
    def _():
        zeros16 = jnp.zeros((16,), jnp.int32)
        neg1 = jnp.full((16,), -1, jnp.int32)
        ones16 = jnp.full((16,), 1, jnp.int32)

        def z(k, _):
            supp_v[pl.ds(k * 16, 16)] = zeros16
            return 0

        lax.fori_loop(0, npa // 16, z, 0)

        def f(k, _):
            keep_v[pl.ds(k * 16, 16)] = neg1
            return 0

        lax.fori_loop(0, 1024 // 16, f, 0)

        lanes = lax.iota(jnp.int32, 16)
        lane0 = lanes == 0

        def batch(bi, cnt):
            row0 = bi * _BAT
            pltpu.sync_copy(cand_hbm.at[pl.ds(row0, _BAT)], cand_v)
            pltpu.sync_copy(bits_hbm.at[pl.ds(row0, _BAT)], bits_v)

            def group(g, cnt):
                for l in range(16):
                    r = g * 16 + l
                    i = row0 + r
                    chunk = supp_v[pl.ds(row0 + g * 16, 16)]
                    live = ((chunk[l] == 0) & (cnt < _TOPN) & (i < n))

                    def do(c, r=r, i=i):
                        plsc.store_scatter(keep_v, [jnp.full((16,), c,
                                                            jnp.int32)],
                                           jnp.full((16,), i, jnp.int32),
                                           mask=lane0)
                        for ch in range(_C // 16):
                            cm = cand_v[r, pl.ds(ch * 16, 16)]
                            bm = bits_v[r, pl.ds(ch * 16, 16)]
                            msk = (bm > 0.5) & (cm >= 0)
                            plsc.store_scatter(supp_v, [cm], ones16, mask=msk)
                        return c + 1

                    cnt = lax.cond(live, do, lambda c: c, cnt)
                return cnt

            return lax.fori_loop(0, _BAT // 16, group, cnt)

        def w_cond(c):
            bi, cnt = c
            return (bi < npa // _BAT) & (cnt < _TOPN)

        def w_body(c):
            bi, cnt = c
            return bi + 1, batch(bi, cnt)

        lax.while_loop(w_cond, w_body, (0, 0))
        pltpu.sync_copy(keep_v.at[pl.ds(0, _TOPN)], out_hbm)


def _run(r_boxes, interpret=False):
    n = r_boxes.shape[0]
    npa = ((n + 511) // 512) * 512
    boxes_p = jnp.zeros((npa, 5), jnp.float32).at[:n].set(r_boxes)
    boxes_t = boxes_p.T

    tbl = pl.pallas_call(
        _prep_kernel,
        grid=(1,),
        in_specs=[pl.BlockSpec((5, npa), lambda i: (0, 0))],
        out_specs=pl.BlockSpec((8, npa), lambda i: (0, 0)),
        out_shape=jax.ShapeDtypeStruct((8, npa), jnp.float32),
        interpret=interpret,
    )(boxes_t)
    tblt = tbl.T

    wpr = npa // 16
    wnp = np.zeros((_P1C, _P1C // 16), np.float32)
    for cc in range(_P1C):
        wnp[cc, cc // 16] = float(1 << (cc % 16))
    wmat = jnp.asarray(wnp)
    m0p = pl.pallas_call(
        functools.partial(_p1_kernel, n, npa),
        grid=(npa // _P1R,),
        in_specs=[pl.BlockSpec((_P1R, 8), lambda i: (i, 0)),
                  pl.BlockSpec((8, npa), lambda i: (0, 0)),
                  pl.BlockSpec((_P1C, _P1C // 16), lambda i: (0, 0))],
        out_specs=pl.BlockSpec((_P1R, wpr), lambda i: (i, 0)),
        out_shape=jax.ShapeDtypeStruct((npa, wpr), jnp.int32),
        compiler_params=pltpu.CompilerParams(
            dimension_semantics=("arbitrary",)),
        interpret=interpret,
    )(tblt, tbl, wmat)

    mesh = plsc.VectorSubcoreMesh(core_axis_name="c", subcore_axis_name="s",
                                  num_cores=2, num_subcores=16)
    cand, cbox = pl.kernel(
        functools.partial(_p2_kernel, npa, n),
        out_type=(jax.ShapeDtypeStruct((npa, _C), jnp.int32),
                  jax.ShapeDtypeStruct((npa, 5 * _C), jnp.float32)),
        mesh=mesh,
        compiler_params=pltpu.CompilerParams(needs_layout_passes=False),
        scratch_types=[pltpu.VMEM((8, npa), jnp.float32),
                       pltpu.VMEM((_SUB, wpr), jnp.int32),
                       pltpu.VMEM((_SUB, _C), jnp.int32),
                       pltpu.VMEM((_SUB, 5 * _C), jnp.float32)],
        interpret=interpret,
    )(tbl, m0p)

    bits = pl.pallas_call(
        _clip_kernel,
        grid=(npa // _CLR,),
        in_specs=[pl.BlockSpec((_CLR, 5), lambda t: (t, 0)),
                  pl.BlockSpec((_CLR, 5 * _C), lambda t: (t, 0)),
                  pl.BlockSpec((_CLR, _C), lambda t: (t, 0))],
        out_specs=pl.BlockSpec((_CLR, _C), lambda t: (t, 0)),
        out_shape=jax.ShapeDtypeStruct((npa, _C), jnp.float32),
        compiler_params=pltpu.CompilerParams(
            dimension_semantics=("arbitrary",)),
        interpret=interpret,
    )(boxes_p, cbox, cand)

    keep_idx = pl.kernel(
        functools.partial(_p4_kernel, npa, n),
        out_type=jax.ShapeDtypeStruct((_TOPN,), jnp.int32),
        mesh=mesh,
        compiler_params=pltpu.CompilerParams(needs_layout_passes=False),
        scratch_types=[pltpu.VMEM((npa,), jnp.int32),
                       pltpu.VMEM((_BAT, _C), jnp.int32),
                       pltpu.VMEM((_BAT, _C), jnp.float32),
                       pltpu.VMEM((1024,), jnp.int32)],
        interpret=interpret,
    )(cand, bits)

    return keep_idx.astype(jnp.int64)


def kernel(r_boxes):
    return _run(r_boxes)


# speedup vs baseline: 1771.1689x; 1.3836x over previous
import functools

import jax
import jax.numpy as jnp
import numpy as np
from jax import lax
from jax.experimental import pallas as pl
from jax.experimental.pallas import tpu as pltpu
from jax.experimental.pallas import tpu_sc as plsc

_THR = 0.7
_TOPN = 1000
_EPS = 1e-8
_C = 128
_P1R = 64
_P1C = 512
_NW = 32
_SUB = 16
_BAT = 256
_CLR = 32


def _corners(xc, yc, w, h, th):
    c = jnp.cos(th)
    s = jnp.sin(th)
    dx = w * 0.5
    dy = h * 0.5
    xs, ys = [], []
    for lx, ly in ((-1.0, -1.0), (1.0, -1.0), (1.0, 1.0), (-1.0, 1.0)):
        ox = lx * dx
        oy = ly * dy
        xs.append(xc + ox * c - oy * s)
        ys.append(yc + ox * s + oy * c)
    return xs, ys


def _dir_area(px, py, qx, qy):
    ex = [qx[(p + 1) % 4] - qx[p] for p in range(4)]
    ey = [qy[(p + 1) % 4] - qy[p] for p in range(4)]
    cst = [ex[p] * qy[p] - ey[p] * qx[p] for p in range(4)]
    d = [[ex[p] * py[v] - ey[p] * px[v] - cst[p] for p in range(4)]
         for v in range(4)]
    total = None
    for k in range(4):
        k1 = (k + 1) % 4
        t0 = jnp.zeros_like(d[0][0])
        t1 = jnp.ones_like(d[0][0])
        empty = None
        for p in range(4):
            da, db = d[k][p], d[k1][p]
            denom = da - db
            t = da / jnp.where(denom == 0.0, 1.0, denom)
            t0 = jnp.where((da < 0.0) & (db >= 0.0), jnp.maximum(t0, t), t0)
            t1 = jnp.where((da >= 0.0) & (db < 0.0), jnp.minimum(t1, t), t1)
            both_out = (da < 0.0) & (db < 0.0)
            empty = both_out if empty is None else (empty | both_out)
        t1 = jnp.maximum(t1, t0)
        span = jnp.where(empty, 0.0, t1 - t0)
        contrib = (py[k1] - py[k]) * (
            px[k] * span + (px[k1] - px[k]) * 0.5 * span * (t1 + t0))
        total = contrib if total is None else total + contrib
    return total


def _prep_kernel(bt_ref, tbl_ref):
    bt = bt_ref[...]
    xc, yc = bt[0:1], bt[1:2]
    w, h, th = bt[2:3], bt[3:4], bt[4:5]
    c, s = jnp.cos(th), jnp.sin(th)
    extx = jnp.abs(w * 0.5 * c) + jnp.abs(h * 0.5 * s)
    exty = jnp.abs(w * 0.5 * s) + jnp.abs(h * 0.5 * c)
    tbl_ref[...] = jnp.concatenate([xc, yc, w, h, th, extx, exty, w * h],
                                   axis=0)


def _p1_kernel(n, npa, tblt_ref, tbl_ref, wmat_ref, out_ref):
    ib = pl.program_id(0)
    wmat = wmat_ref[...]
    tr = tblt_ref[...]
    xr, yr = tr[:, 0:1], tr[:, 1:2]
    exr, eyr, ar = tr[:, 5:6], tr[:, 6:7], tr[:, 7:8]
    pieces = []
    for jt in range(npa // _P1C):
        tc = tbl_ref[:, pl.ds(jt * _P1C, _P1C)]
        xc_, yc_ = tc[0:1, :], tc[1:2, :]
        exc, eyc, ac = tc[5:6, :], tc[6:7, :], tc[7:8, :]
        ox = (jnp.minimum(xr + exr, xc_ + exc)
              - jnp.maximum(xr - exr, xc_ - exc))
        oy = (jnp.minimum(yr + eyr, yc_ + eyc)
              - jnp.maximum(yr - eyr, yc_ - eyc))
        oxp = jnp.maximum(ox, 0.0)
        oyp = jnp.maximum(oy, 0.0)
        amin = jnp.minimum(ar, ac)
        amax = jnp.maximum(ar, ac)
        good = (oxp * oyp >= 0.41 * (ar + ac)) & (amin >= 0.699 * amax)
        row_id = ib * _P1R + lax.broadcasted_iota(jnp.int32, (_P1R, _P1C), 0)
        col_id = jt * _P1C + lax.broadcasted_iota(jnp.int32, (_P1R, _P1C), 1)
        good = good & (col_id > row_id) & (col_id < n)
        pieces.append(jnp.dot(jnp.where(good, 1.0, 0.0), wmat,
                              preferred_element_type=jnp.float32))
    out_ref[...] = jnp.concatenate(pieces, axis=1).astype(jnp.int32)


def _p2_kernel(npa, n, tbl_hbm, m0p_hbm, cand_hbm, cbox_hbm,
               tbl_v, w_v, cand_v, cbox_v):
    wid = lax.axis_index("s") * 2 + lax.axis_index("c")
    half_w = npa // (2 * _NW)
    nsb = half_w // _SUB
    wpr = npa // 16
    pltpu.sync_copy(tbl_hbm, tbl_v)
    lanes = lax.iota(jnp.int32, 16)
    neg1 = jnp.full((16,), -1, jnp.int32)

    def sub_batch(sb, _):
        top = sb < nsb
        row0 = jnp.where(top, wid * half_w + sb * _SUB,
                         npa - (wid + 1) * half_w + (sb - nsb) * _SUB)
        pltpu.sync_copy(m0p_hbm.at[pl.ds(row0, _SUB)], w_v)

        def fill(r, _):
            for ch in range(_C // 16):
                cand_v[r, pl.ds(ch * 16, 16)] = neg1
            return 0

        lax.fori_loop(0, _SUB, fill, 0)

        def row_body(r, _):
            i_row = row0 + r
            rspl = jnp.full((16,), r, jnp.int32)
            base0 = jnp.zeros((16,), jnp.int32)

            def chunk_body(wc, base):
                words = w_v[r, pl.ds(wc * 16, 16)]
                any_w = jnp.max(words)

                def nonempty(base):
                    for l in range(16):
                        wscal = words[l]

                        def have(b, wscal=wscal, l=l):
                            wspl = jnp.full((16,), wscal, jnp.int32)
                            mask = ((wspl >> lanes) & 1) == 1
                            j_ids = (wc * 16 + l) * 16 + lanes
                            cum = plsc.cumsum(mask.astype(jnp.int32))
                            idx = b + cum - 1
                            mask2 = mask & (idx < _C)
                            plsc.store_scatter(cand_v, [rspl, idx], j_ids,
                                               mask=mask2)
                            for p in range(5):
                                pspl = jnp.full((16,), p, jnp.int32)
                                vals = plsc.load_gather(tbl_v, [pspl, j_ids],
                                                        mask=mask2)
                                plsc.store_scatter(cbox_v,
                                                   [rspl, idx + (p * _C)],
                                                   vals, mask=mask2)
                            return b + plsc.all_reduce_population_count(mask)

                        base = lax.cond(wscal != 0, have, lambda b: b, base)
                    return base

                return lax.cond(any_w > 0, nonempty, lambda b: b, base)

            lax.fori_loop(i_row >> 8, wpr // 16, chunk_body, base0)
            return 0

        lax.fori_loop(0, _SUB, row_body, 0)
        pltpu.sync_copy(cand_v, cand_hbm.at[pl.ds(row0, _SUB)])
        pltpu.sync_copy(cbox_v, cbox_hbm.at[pl.ds(row0, _SUB)])
        return 0

    lax.fori_loop(0, 2 * nsb, sub_batch, 0)


def _clip_kernel(boxes_ref, cbox_ref, cand_ref, bits_ref):
    b = boxes_ref[...]
    xc_r, yc_r = b[:, 0:1], b[:, 1:2]
    w_r, h_r, th_r = b[:, 2:3], b[:, 3:4], b[:, 4:5]
    cb = cbox_ref[...]
    xc_c, yc_c = cb[:, 0:_C], cb[:, _C:2 * _C]
    w_c, h_c = cb[:, 2 * _C:3 * _C], cb[:, 3 * _C:4 * _C]
    th_c = cb[:, 4 * _C:5 * _C]
    rx, ry = _corners(xc_r, yc_r, w_r, h_r, th_r)
    cx, cy = _corners(xc_c, yc_c, w_c, h_c, th_c)
    inter = _dir_area(rx, ry, cx, cy) + _dir_area(cx, cy, rx, ry)
    iou = inter / (w_r * h_r + w_c * h_c - inter + _EPS)
    cand = cand_ref[...]
    bits_ref[...] = jnp.where((cand >= 0) & (iou >= _THR), 1.0, 0.0)


def _p4_kernel(npa, n, cand_hbm, bits_hbm, out_hbm,
               supp_v, cand_v, bits_v, keep_v):
    wid = lax.axis_index("s") * 2 + lax.axis_index("c")

    @pl.when(wid == 0)
    def _():
        zeros16 = jnp.zeros((16,), jnp.int32)
        neg1 = jnp.full((16,), -1, jnp.int32)
        ones16 = jnp.full((16,), 1, jnp.int32)

        def z(k, _):
            supp_v[pl.ds(k * 16, 16)] = zeros16
            return 0

        lax.fori_loop(0, npa // 16, z, 0)

        def f(k, _):
            keep_v[pl.ds(k * 16, 16)] = neg1
            return 0

        lax.fori_loop(0, 1024 // 16, f, 0)

        lanes = lax.iota(jnp.int32, 16)
        lane0 = lanes == 0

        def batch(bi, cnt):
            row0 = bi * _BAT
            pltpu.sync_copy(cand_hbm.at[pl.ds(row0, _BAT)], cand_v)
            pltpu.sync_copy(bits_hbm.at[pl.ds(row0, _BAT)], bits_v)

            def group(g, cnt):
                for l in range(16):
                    r = g * 16 + l
                    i = row0 + r
                    chunk = supp_v[pl.ds(row0 + g * 16, 16)]
                    live = ((chunk[l] == 0) & (cnt < _TOPN) & (i < n))

                    def do(c, r=r, i=i):
                        plsc.store_scatter(keep_v, [jnp.full((16,), c,
                                                            jnp.int32)],
                                           jnp.full((16,), i, jnp.int32),
                                           mask=lane0)
                        for ch in range(_C // 16):
                            cm = cand_v[r, pl.ds(ch * 16, 16)]
                            bm = bits_v[r, pl.ds(ch * 16, 16)]
                            msk = (bm > 0.5) & (cm >= 0)
                            plsc.store_scatter(supp_v, [cm], ones16, mask=msk)
                        return c + 1

                    cnt = lax.cond(live, do, lambda c: c, cnt)
                return cnt

            return lax.fori_loop(0, _BAT // 16, group, cnt)

        def w_cond(c):
            bi, cnt = c
            return (bi < npa // _BAT) & (cnt < _TOPN)

        def w_body(c):
            bi, cnt = c
            return bi + 1, batch(bi, cnt)

        lax.while_loop(w_cond, w_body, (0, 0))
        pltpu.sync_copy(keep_v.at[pl.ds(0, _TOPN)], out_hbm)


def _run(r_boxes, interpret=False):
    n = r_boxes.shape[0]
    npa = ((n + 511) // 512) * 512
    boxes_p = jnp.zeros((npa, 5), jnp.float32).at[:n].set(r_boxes)
    boxes_t = boxes_p.T

    tbl = pl.pallas_call(
        _prep_kernel,
        grid=(1,),
        in_specs=[pl.BlockSpec((5, npa), lambda i: (0, 0))],
        out_specs=pl.BlockSpec((8, npa), lambda i: (0, 0)),
        out_shape=jax.ShapeDtypeStruct((8, npa), jnp.float32),
        interpret=interpret,
    )(boxes_t)
    tblt = tbl.T

    wpr = npa // 16
    wnp = np.zeros((_P1C, _P1C // 16), np.float32)
    for cc in range(_P1C):
        wnp[cc, cc // 16] = float(1 << (cc % 16))
    wmat = jnp.asarray(wnp)
    m0p = pl.pallas_call(
        functools.partial(_p1_kernel, n, npa),
        grid=(npa // _P1R,),
        in_specs=[pl.BlockSpec((_P1R, 8), lambda i: (i, 0)),
                  pl.BlockSpec((8, npa), lambda i: (0, 0)),
                  pl.BlockSpec((_P1C, _P1C // 16), lambda i: (0, 0))],
        out_specs=pl.BlockSpec((_P1R, wpr), lambda i: (i, 0)),
        out_shape=jax.ShapeDtypeStruct((npa, wpr), jnp.int32),
        compiler_params=pltpu.CompilerParams(
            dimension_semantics=("arbitrary",)),
        interpret=interpret,
    )(tblt, tbl, wmat)

    mesh = plsc.VectorSubcoreMesh(core_axis_name="c", subcore_axis_name="s",
                                  num_cores=2, num_subcores=16)
    cand, cbox = pl.kernel(
        functools.partial(_p2_kernel, npa, n),
        out_type=(jax.ShapeDtypeStruct((npa, _C), jnp.int32),
                  jax.ShapeDtypeStruct((npa, 5 * _C), jnp.float32)),
        mesh=mesh,
        compiler_params=pltpu.CompilerParams(needs_layout_passes=False),
        scratch_types=[pltpu.VMEM((8, npa), jnp.float32),
                       pltpu.VMEM((_SUB, wpr), jnp.int32),
                       pltpu.VMEM((_SUB, _C), jnp.int32),
                       pltpu.VMEM((_SUB, 5 * _C), jnp.float32)],
        interpret=interpret,
    )(tbl, m0p)

    bits = pl.pallas_call(
        _clip_kernel,
        grid=(npa // _CLR,),
        in_specs=[pl.BlockSpec((_CLR, 5), lambda t: (t, 0)),
                  pl.BlockSpec((_CLR, 5 * _C), lambda t: (t, 0)),
                  pl.BlockSpec((_CLR, _C), lambda t: (t, 0))],
        out_specs=pl.BlockSpec((_CLR, _C), lambda t: (t, 0)),
        out_shape=jax.ShapeDtypeStruct((npa, _C), jnp.float32),
        compiler_params=pltpu.CompilerParams(
            dimension_semantics=("arbitrary",)),
        interpret=interpret,
    )(boxes_p, cbox, cand)

    keep_idx = pl.kernel(
        functools.partial(_p4_kernel, npa, n),
        out_type=jax.ShapeDtypeStruct((_TOPN,), jnp.int32),
        mesh=mesh,
        compiler_params=pltpu.CompilerParams(needs_layout_passes=False),
        scratch_types=[pltpu.VMEM((npa,), jnp.int32),
                       pltpu.VMEM((_BAT, _C), jnp.int32),
                       pltpu.VMEM((_BAT, _C), jnp.float32),
                       pltpu.VMEM((1024,), jnp.int32)],
        interpret=interpret,
    )(cand, bits)

    return keep_idx.astype(jnp.int64)


def kernel(r_boxes):
    return _run(r_boxes)
